# baseline (speedup 1.0000x reference)
import numpy as np

import jax
import jax.numpy as jnp
from jax import lax
from jax.experimental import pallas as pl
from jax.experimental.pallas import tpu as pltpu
from jax.experimental.pallas import tpu_sc as plsc

N = 10000
D = 128
H = 2
AH = 16
E = 320000
R = 144
RW = 72
CH = 64
NCHUNK = E // CH
NZC = N // CH
ZTAIL = N - NZC * CH
BN = 1000

_HIGH = lax.Precision.HIGHEST

_SIGMA = np.empty(D, dtype=np.int32)
for _k in range(D // 32):
    for _m in range(16):
        _SIGMA[32 * _k + 2 * _m] = 32 * _k + _m
        _SIGMA[32 * _k + 2 * _m + 1] = 32 * _k + 16 + _m


def _prep_body(x_ref, W1_ref, b1_ref, W2_ref, b2_ref, A1s_ref, A1d_ref,
               a1_ref, S_ref, P_ref, F_ref):
    h = pl.program_id(0)
    xb = x_ref[...]
    f = jnp.maximum(jnp.dot(xb, W1_ref[0], precision=_HIGH) + b1_ref[h], 0.0)
    f = jnp.dot(f, W2_ref[0], precision=_HIGH) + b2_ref[h]
    S_ref[0] = jnp.dot(f, A1s_ref[0], precision=_HIGH) + a1_ref[h]
    P_ref[0] = jnp.dot(f, A1d_ref[0], precision=_HIGH)
    F_ref[0] = f


def _prep(x, W1, b1, W2, b2, A1s, A1d, a1):
    grid = (H, N // BN)
    return pl.pallas_call(
        _prep_body,
        grid=grid,
        in_specs=[
            pl.BlockSpec((BN, D), lambda h, i: (i, 0)),
            pl.BlockSpec((1, D, D), lambda h, i: (h, 0, 0)),
            pl.BlockSpec((H, D), lambda h, i: (0, 0)),
            pl.BlockSpec((1, D, D), lambda h, i: (h, 0, 0)),
            pl.BlockSpec((H, D), lambda h, i: (0, 0)),
            pl.BlockSpec((1, D, AH), lambda h, i: (h, 0, 0)),
            pl.BlockSpec((1, D, AH), lambda h, i: (h, 0, 0)),
            pl.BlockSpec((H, AH), lambda h, i: (0, 0)),
        ],
        out_specs=[
            pl.BlockSpec((1, BN, AH), lambda h, i: (h, i, 0)),
            pl.BlockSpec((1, BN, AH), lambda h, i: (h, i, 0)),
            pl.BlockSpec((1, BN, D), lambda h, i: (h, i, 0)),
        ],
        out_shape=[
            jax.ShapeDtypeStruct((H, N, AH), jnp.float32),
            jax.ShapeDtypeStruct((H, N, AH), jnp.float32),
            jax.ShapeDtypeStruct((H, N, D), jnp.float32),
        ],
    )(x, W1, b1, W2, b2, A1s, A1d, a1)


NISL = 3
NGSL = 2
NITER = (NCHUNK + 15) // 16
NUNROLL = 6
NOUTER = (NITER + NUNROLL) // NUNROLL


def _edge_body(S_hbm, T_hbm, srcr_hbm, sadj_hbm, dadj_hbm, elem_hbm,
               consts_hbm, out_hbm, acc, cbuf, isrc, iga, igb, elv, g1v,
               gfv, outv, scv, semA, semB, semS):
    h = lax.axis_index("c")
    tid = lax.axis_index("s")
    hN = h * N
    hE = h * E

    pltpu.sync_copy(consts_hbm.at[h], cbuf)
    c_vec = cbuf[0]
    a2v = cbuf[1]
    a2rep = cbuf[2]

    def _zrow(i, _):
        for k in range(R // 16):
            outv[0][i, pl.ds(k * 16, 16)] = jnp.zeros((16,), jnp.float32)
        return 0

    lax.fori_loop(0, CH, _zrow, 0)

    def _zchunk(j, _):
        z = j * 16 + tid

        @pl.when(z < NZC)
        def _():
            pltpu.sync_copy(outv[0], acc.at[pl.ds(z * CH, CH)])

        if ZTAIL:
            @pl.when(z == NZC)
            def _():
                pltpu.sync_copy(outv[0].at[pl.ds(0, ZTAIL)],
                                acc.at[pl.ds(NZC * CH, ZTAIL)])

        return 0

    lax.fori_loop(0, (NZC + 16) // 16, _zchunk, 0)
    plsc.subcore_barrier()

    def _valid(j):
        return (j * 16 + tid) < NCHUNK

    def _base(j):
        return (j * 16 + tid) * CH

    def _issue_a(j, s):
        b = _base(j)
        pltpu.async_copy(srcr_hbm.at[pl.ds(b, CH)], isrc[s], semA[s])
        pltpu.async_copy(sadj_hbm.at[pl.ds(hE + b, CH)], iga[s], semA[s])
        pltpu.async_copy(dadj_hbm.at[pl.ds(hE + b, CH)], igb[s], semA[s])
        pltpu.async_copy(elem_hbm.at[pl.ds(b, CH)],
                         elv[s].at[pl.ds(0, CH)], semA[s])

    def _wait_a(s):
        pltpu.make_async_copy(srcr_hbm.at[pl.ds(0, CH)], isrc[s], semA[s]).wait()
        pltpu.make_async_copy(sadj_hbm.at[pl.ds(0, CH)], iga[s], semA[s]).wait()
        pltpu.make_async_copy(dadj_hbm.at[pl.ds(0, CH)], igb[s], semA[s]).wait()
        pltpu.make_async_copy(elem_hbm.at[pl.ds(0, CH)],
                              elv[s].at[pl.ds(0, CH)], semA[s]).wait()

    def _issue_b(si, sg):
        pltpu.async_copy(S_hbm.at[iga[si]], g1v[sg], semB[sg])
        pltpu.async_copy(T_hbm.at[igb[si]], gfv[sg], semB[sg])

    def _wait_b(sg):
        pltpu.make_async_copy(S_hbm.at[pl.ds(0, CH)], g1v[sg], semB[sg]).wait()
        pltpu.make_async_copy(T_hbm.at[pl.ds(0, CH)], gfv[sg], semB[sg]).wait()

    def _issue_s(si):
        pltpu.async_copy(outv[si], acc.at[isrc[si]], semS[si], add=True)

    def _wait_s(si):
        pltpu.make_async_copy(outv[si], acc.at[pl.ds(0, CH)], semS[si]).wait()

    lane = lax.iota(jnp.int32, 16)

    def _compute(sg, si):
        def _group(g, _):
            g16 = g * 16
            el = elv[si][pl.ds(g16, 16)]
            row = lane + g16
            sc = a2rep
            for p in range(AH // 2):
                w = plsc.load_gather(gfv[sg],
                                     [row, jnp.full((16,), D // 2 + p,
                                                    jnp.int32)])
                pd0 = plsc.bitcast(w << 16, jnp.float32)
                pd1 = plsc.bitcast(w & -65536, jnp.float32)
                u0 = (plsc.load_gather(g1v[sg],
                                       [row, jnp.full((16,), 2 * p,
                                                      jnp.int32)])
                      + pd0 + el * c_vec[2 * p])
                u1 = (plsc.load_gather(g1v[sg],
                                       [row, jnp.full((16,), 2 * p + 1,
                                                      jnp.int32)])
                      + pd1 + el * c_vec[2 * p + 1])
                sc = (sc + jnp.maximum(u0, 0.0) * a2v[2 * p]
                      + jnp.maximum(u1, 0.0) * a2v[2 * p + 1])
            ev = jnp.exp(jnp.where(sc > 0, sc, 0.2 * sc))
            scv[pl.ds(g16, 16)] = ev
            return 0

        lax.fori_loop(0, CH // 16, _group, 0)

        def _scale(i, _):
            e = scv[pl.ds(i, 16)][0]
            for k in range(D // 32):
                w = gfv[sg][i, pl.ds(k * 16, 16)]
                pe = plsc.bitcast(w << 16, jnp.float32) * e
                po = plsc.bitcast(w & -65536, jnp.float32) * e
                outv[si][i, pl.ds(k * 32, 16)] = pe
                outv[si][i, pl.ds(k * 32 + 16, 16)] = po
            outv[si][i, pl.ds(D, 16)] = jnp.where(lane == 0, e, 0.0)
            return 0

        lax.fori_loop(0, CH, _scale, 0)

    _issue_a(0, 0)
    _issue_a(1, 1)
    _wait_a(0)
    _issue_b(0, 0)

    def _iter(j, si, si1, si2, sg, sg1):
        @pl.when(_valid(j))
        def _():
            _wait_b(sg)

        @pl.when(_valid(j + 1))
        def _():
            _wait_a(si1)
            _issue_b(si1, sg1)

        @pl.when(_valid(j))
        def _():
            _compute(sg, si)
            _issue_s(si)

        @pl.when((j >= 1) & _valid(j - 1))
        def _():
            _wait_s(si2)

        @pl.when(_valid(j + 2))
        def _():
            _issue_a(j + 2, si2)

    def _outer(jj, _):
        j0 = jj * NUNROLL
        for b in range(NUNROLL):
            _iter(j0 + b, b % NISL, (b + 1) % NISL, (b + 2) % NISL,
                  b % NGSL, (b + 1) % NGSL)
        return 0

    lax.fori_loop(0, NOUTER, _outer, 0)
    plsc.subcore_barrier()

    def _dump(j, _):
        z = j * 16 + tid

        @pl.when(z < NZC)
        def _():
            pltpu.sync_copy(acc.at[pl.ds(z * CH, CH)], outv[0])
            pltpu.sync_copy(outv[0], out_hbm.at[pl.ds(hN + z * CH, CH)])

        if ZTAIL:
            @pl.when(z == NZC)
            def _():
                pltpu.sync_copy(acc.at[pl.ds(NZC * CH, ZTAIL)],
                                outv[0].at[pl.ds(0, ZTAIL)])
                pltpu.sync_copy(outv[0].at[pl.ds(0, ZTAIL)],
                                out_hbm.at[pl.ds(hN + NZC * CH, ZTAIL)])

        return 0

    lax.fori_loop(0, (NZC + 16) // 16, _dump, 0)


def _edge_pass(S2, T2, srcr, sadj2, dadj2, elem, consts):
    mesh = plsc.VectorSubcoreMesh(core_axis_name="c", subcore_axis_name="s")
    k = pl.kernel(
        _edge_body,
        out_type=jax.ShapeDtypeStruct((H * N, R), jnp.float32),
        mesh=mesh,
        compiler_params=pltpu.CompilerParams(
            needs_layout_passes=False, use_tc_tiling_on_sc=False),
        scratch_types=[
            pltpu.VMEM_SHARED((N, R), jnp.float32),
            pltpu.VMEM((4, 16), jnp.float32),
            tuple(pltpu.VMEM((CH,), jnp.int32) for _ in range(NISL)),
            tuple(pltpu.VMEM((CH,), jnp.int32) for _ in range(NISL)),
            tuple(pltpu.VMEM((CH,), jnp.int32) for _ in range(NISL)),
            tuple(pltpu.VMEM((CH + 16,), jnp.float32) for _ in range(NISL)),
            tuple(pltpu.VMEM((CH, AH), jnp.float32) for _ in range(NGSL)),
            tuple(pltpu.VMEM((CH, RW), jnp.int32) for _ in range(NGSL)),
            tuple(pltpu.VMEM((CH, R), jnp.float32) for _ in range(NISL)),
            pltpu.VMEM((CH + 16,), jnp.float32),
            tuple(pltpu.SemaphoreType.DMA for _ in range(NISL)),
            tuple(pltpu.SemaphoreType.DMA for _ in range(NGSL)),
            tuple(pltpu.SemaphoreType.DMA for _ in range(NISL)),
        ],
    )
    return k(S2, T2, srcr, sadj2, dadj2, elem, consts)


def _fin_body(P_ref, o_ref):
    pb = P_ref[0]
    o_ref[...] = pb[:, :D] / (pb[:, D:D + 1] + 1e-10)


def _finalize(P):
    grid = (H, N // BN)
    return pl.pallas_call(
        _fin_body,
        grid=grid,
        in_specs=[pl.BlockSpec((1, BN, R), lambda h, i: (h, i, 0))],
        out_specs=pl.BlockSpec((BN, D), lambda h, i: (i, h)),
        out_shape=jax.ShapeDtypeStruct((N, H * D), jnp.float32),
    )(P)


def kernel(x, idx, elem, W1, b1, W2, b2, A1, a1, A2, a2):
    A1s = A1[:, :D, :]
    A1d = A1[:, D:2 * D, :]
    c = A1[:, 2 * D, :]
    a2v = A2[:, :, 0]
    a2rep = jnp.broadcast_to(a2, (H, 16))
    consts = jnp.stack([c, a2v, a2rep, jnp.zeros_like(c)], axis=1)

    S, Pd, F = _prep(x, W1, b1, W2, b2, A1s, A1d, a1)
    S2 = S.reshape(H * N, AH)
    Tbf = jnp.concatenate([F[:, :, _SIGMA], Pd], axis=2).astype(jnp.bfloat16)
    T2 = lax.bitcast_convert_type(
        Tbf.reshape(H * N, RW, 2), jnp.int32)
    src = idx[0]
    dst = idx[1]
    sadj2 = jnp.concatenate([src, src + N]).astype(jnp.int32)
    dadj2 = jnp.concatenate([dst, dst + N]).astype(jnp.int32)
    P = _edge_pass(S2, T2, src, sadj2, dadj2, elem, consts)
    return _finalize(P.reshape(H, N, R))


# speedup vs baseline: 1.7748x; 1.7748x over previous
import functools

import jax
import jax.numpy as jnp
from jax import lax
from jax.experimental import pallas as pl
from jax.experimental.pallas import tpu as pltpu
from jax.experimental.pallas import tpu_sc as plsc

N = 10000
D = 128
H = 2
AH = 16
E = 320000
R = 144
CH = 80
NCHUNK = E // CH
NZC = N // CH
ZTAIL = N - NZC * CH
BN = 1000

_HIGH = lax.Precision.HIGHEST


def _prep_body(x_ref, W1_ref, b1_ref, W2_ref, b2_ref, A1s_ref, A1d_ref,
               a1_ref, S_ref, T_ref):
    h = pl.program_id(0)
    xb = x_ref[...]
    f = jnp.maximum(jnp.dot(xb, W1_ref[0], precision=_HIGH) + b1_ref[h], 0.0)
    f = jnp.dot(f, W2_ref[0], precision=_HIGH) + b2_ref[h]
    S_ref[0] = jnp.dot(f, A1s_ref[0], precision=_HIGH) + a1_ref[h]
    pd = jnp.dot(f, A1d_ref[0], precision=_HIGH)
    T_ref[0] = jnp.concatenate([f, pd], axis=1)


def _prep(x, W1, b1, W2, b2, A1s, A1d, a1):
    grid = (H, N // BN)
    return pl.pallas_call(
        _prep_body,
        grid=grid,
        in_specs=[
            pl.BlockSpec((BN, D), lambda h, i: (i, 0)),
            pl.BlockSpec((1, D, D), lambda h, i: (h, 0, 0)),
            pl.BlockSpec((H, D), lambda h, i: (0, 0)),
            pl.BlockSpec((1, D, D), lambda h, i: (h, 0, 0)),
            pl.BlockSpec((H, D), lambda h, i: (0, 0)),
            pl.BlockSpec((1, D, AH), lambda h, i: (h, 0, 0)),
            pl.BlockSpec((1, D, AH), lambda h, i: (h, 0, 0)),
            pl.BlockSpec((H, AH), lambda h, i: (0, 0)),
        ],
        out_specs=[
            pl.BlockSpec((1, BN, AH), lambda h, i: (h, i, 0)),
            pl.BlockSpec((1, BN, R), lambda h, i: (h, i, 0)),
        ],
        out_shape=[
            jax.ShapeDtypeStruct((H, N, AH), jnp.float32),
            jax.ShapeDtypeStruct((H, N, R), jnp.float32),
        ],
    )(x, W1, b1, W2, b2, A1s, A1d, a1)


NISL = 4
NGSL = 3
NITER = (NCHUNK + 15) // 16
NUNROLL = 12
NOUTER = (NITER + NUNROLL) // NUNROLL


def _edge_body(S_hbm, T_hbm, srcr_hbm, sadj_hbm, dadj_hbm, elem_hbm,
               consts_hbm, out_hbm, acc, cbuf, isrc, iga, igb, elv, g1v,
               g2v, scv, semA, semB, semS):
    h = lax.axis_index("c")
    tid = lax.axis_index("s")
    hN = h * N
    hE = h * E

    pltpu.sync_copy(consts_hbm.at[h], cbuf)
    c_vec = cbuf[0]
    a2v = cbuf[1]
    a2rep = cbuf[2]

    def _zrow(i, _):
        for k in range(R // 16):
            g2v[0][i, pl.ds(k * 16, 16)] = jnp.zeros((16,), jnp.float32)
        return 0

    lax.fori_loop(0, CH, _zrow, 0)

    def _zchunk(j, _):
        z = j * 16 + tid

        @pl.when(z < NZC)
        def _():
            pltpu.sync_copy(g2v[0], acc.at[pl.ds(z * CH, CH)])

        if ZTAIL:
            @pl.when(z == NZC)
            def _():
                pltpu.sync_copy(g2v[0].at[pl.ds(0, ZTAIL)],
                                acc.at[pl.ds(NZC * CH, ZTAIL)])

        return 0

    lax.fori_loop(0, (NZC + 16) // 16, _zchunk, 0)
    plsc.subcore_barrier()

    def _valid(j):
        return (j * 16 + tid) < NCHUNK

    def _base(j):
        return (j * 16 + tid) * CH

    def _issue_a(j, s):
        b = _base(j)
        pltpu.async_copy(srcr_hbm.at[pl.ds(b, CH)], isrc[s], semA[s])
        pltpu.async_copy(sadj_hbm.at[pl.ds(hE + b, CH)], iga[s], semA[s])
        pltpu.async_copy(dadj_hbm.at[pl.ds(hE + b, CH)], igb[s], semA[s])
        pltpu.async_copy(elem_hbm.at[pl.ds(b, CH)],
                         elv[s].at[pl.ds(0, CH)], semA[s])

    def _wait_a(s):
        pltpu.make_async_copy(srcr_hbm.at[pl.ds(0, CH)], isrc[s], semA[s]).wait()
        pltpu.make_async_copy(sadj_hbm.at[pl.ds(0, CH)], iga[s], semA[s]).wait()
        pltpu.make_async_copy(dadj_hbm.at[pl.ds(0, CH)], igb[s], semA[s]).wait()
        pltpu.make_async_copy(elem_hbm.at[pl.ds(0, CH)],
                              elv[s].at[pl.ds(0, CH)], semA[s]).wait()

    def _issue_b(si, sg):
        pltpu.async_copy(S_hbm.at[iga[si]], g1v[sg], semB[sg])
        pltpu.async_copy(T_hbm.at[igb[si]], g2v[sg], semB[sg])

    def _wait_b(sg):
        pltpu.make_async_copy(S_hbm.at[pl.ds(0, CH)], g1v[sg], semB[sg]).wait()
        pltpu.make_async_copy(T_hbm.at[pl.ds(0, CH)], g2v[sg], semB[sg]).wait()

    def _issue_s(sg, si):
        pltpu.async_copy(g2v[sg], acc.at[isrc[si]], semS[sg], add=True)

    def _wait_s(sg):
        pltpu.make_async_copy(g2v[sg], acc.at[pl.ds(0, CH)], semS[sg]).wait()

    lane = lax.iota(jnp.int32, 16)

    def _compute(sg, si):
        def _group(g, _):
            g16 = g * 16
            el = elv[si][pl.ds(g16, 16)]
            row = lane + g16
            sc = a2rep
            for jj in range(AH):
                cj = jnp.full((16,), jj, jnp.int32)
                u = (plsc.load_gather(g1v[sg], [row, cj])
                     + plsc.load_gather(g2v[sg], [row, cj + D])
                     + el * c_vec[jj])
                sc = sc + jnp.maximum(u, 0.0) * a2v[jj]
            ev = jnp.exp(jnp.where(sc > 0, sc, 0.2 * sc))
            scv[pl.ds(g16, 16)] = ev
            return 0

        lax.fori_loop(0, CH // 16, _group, 0)

        def _scale(i, _):
            e = scv[pl.ds(i, 16)][0]
            for k in range(D // 16):
                sl = pl.ds(k * 16, 16)
                g2v[sg][i, sl] = g2v[sg][i, sl] * e
            g2v[sg][i, pl.ds(D, 16)] = jnp.where(lane == 0, e, 0.0)
            return 0

        lax.fori_loop(0, CH, _scale, 0)

    _issue_a(0, 0)
    _issue_a(1, 1)
    _issue_a(2, 2)
    _wait_a(0)
    _issue_b(0, 0)
    _wait_a(1)
    _issue_b(1, 1)

    def _iter(j, si, sg):
        @pl.when(_valid(j))
        def _():
            _wait_b(sg)
            _compute(sg, si)
            _issue_s(sg, si)

        @pl.when((j >= 1) & _valid(j - 1))
        def _():
            _wait_s((sg + 2) % NGSL)

        @pl.when(_valid(j + 3))
        def _():
            _issue_a(j + 3, (si + 3) % NISL)

        @pl.when(_valid(j + 2))
        def _():
            _wait_a((si + 2) % NISL)
            _issue_b((si + 2) % NISL, (sg + 2) % NGSL)

    def _outer(jj, _):
        j0 = jj * NUNROLL
        for b in range(NUNROLL):
            _iter(j0 + b, b % NISL, b % NGSL)
        return 0

    lax.fori_loop(0, NOUTER, _outer, 0)
    plsc.subcore_barrier()

    def _dump(j, _):
        z = j * 16 + tid

        @pl.when(z < NZC)
        def _():
            pltpu.sync_copy(acc.at[pl.ds(z * CH, CH)], g2v[0])
            pltpu.sync_copy(g2v[0], out_hbm.at[pl.ds(hN + z * CH, CH)])

        if ZTAIL:
            @pl.when(z == NZC)
            def _():
                pltpu.sync_copy(acc.at[pl.ds(NZC * CH, ZTAIL)],
                                g2v[0].at[pl.ds(0, ZTAIL)])
                pltpu.sync_copy(g2v[0].at[pl.ds(0, ZTAIL)],
                                out_hbm.at[pl.ds(hN + NZC * CH, ZTAIL)])

        return 0

    lax.fori_loop(0, (NZC + 16) // 16, _dump, 0)


def _edge_pass(S2, T2, srcr, sadj2, dadj2, elem, consts):
    mesh = plsc.VectorSubcoreMesh(core_axis_name="c", subcore_axis_name="s")
    k = pl.kernel(
        _edge_body,
        out_type=jax.ShapeDtypeStruct((H * N, R), jnp.float32),
        mesh=mesh,
        compiler_params=pltpu.CompilerParams(
            needs_layout_passes=False, use_tc_tiling_on_sc=False),
        scratch_types=[
            pltpu.VMEM_SHARED((N, R), jnp.float32),
            pltpu.VMEM((4, 16), jnp.float32),
            tuple(pltpu.VMEM((CH,), jnp.int32) for _ in range(NISL)),
            tuple(pltpu.VMEM((CH,), jnp.int32) for _ in range(NISL)),
            tuple(pltpu.VMEM((CH,), jnp.int32) for _ in range(NISL)),
            tuple(pltpu.VMEM((CH + 16,), jnp.float32) for _ in range(NISL)),
            tuple(pltpu.VMEM((CH, AH), jnp.float32) for _ in range(NGSL)),
            tuple(pltpu.VMEM((CH, R), jnp.float32) for _ in range(NGSL)),
            pltpu.VMEM((CH + 16,), jnp.float32),
            tuple(pltpu.SemaphoreType.DMA for _ in range(NISL)),
            tuple(pltpu.SemaphoreType.DMA for _ in range(NGSL)),
            tuple(pltpu.SemaphoreType.DMA for _ in range(NGSL)),
        ],
    )
    return k(S2, T2, srcr, sadj2, dadj2, elem, consts)


def _fin_body(P_ref, o_ref):
    pb = P_ref[0]
    o_ref[...] = pb[:, :D] / (pb[:, D:D + 1] + 1e-10)


def _finalize(P):
    grid = (H, N // BN)
    return pl.pallas_call(
        _fin_body,
        grid=grid,
        in_specs=[pl.BlockSpec((1, BN, R), lambda h, i: (h, i, 0))],
        out_specs=pl.BlockSpec((BN, D), lambda h, i: (i, h)),
        out_shape=jax.ShapeDtypeStruct((N, H * D), jnp.float32),
    )(P)


def kernel(x, idx, elem, W1, b1, W2, b2, A1, a1, A2, a2):
    A1s = A1[:, :D, :]
    A1d = A1[:, D:2 * D, :]
    c = A1[:, 2 * D, :]
    a2v = A2[:, :, 0]
    a2rep = jnp.broadcast_to(a2, (H, 16))
    consts = jnp.stack([c, a2v, a2rep, jnp.zeros_like(c)], axis=1)

    S, T = _prep(x, W1, b1, W2, b2, A1s, A1d, a1)
    S2 = S.reshape(H * N, AH)
    T2 = T.reshape(H * N, R)
    src = idx[0]
    dst = idx[1]
    sadj2 = jnp.concatenate([src, src + N]).astype(jnp.int32)
    dadj2 = jnp.concatenate([dst, dst + N]).astype(jnp.int32)
    P = _edge_pass(S2, T2, src, sadj2, dadj2, elem, consts)
    return _finalize(P.reshape(H, N, R))


# speedup vs baseline: 1.7922x; 1.0098x over previous
import functools

import jax
import jax.numpy as jnp
from jax import lax
from jax.experimental import pallas as pl
from jax.experimental.pallas import tpu as pltpu
from jax.experimental.pallas import tpu_sc as plsc

N = 10000
D = 128
H = 2
AH = 16
E = 320000
R = 144
CH = 80
NCHUNK = E // CH
NZC = N // CH
ZTAIL = N - NZC * CH
BN = 1000

_HIGH = lax.Precision.HIGHEST


def _prep_body(x_ref, W1_ref, b1_ref, W2_ref, b2_ref, A1s_ref, A1d_ref,
               a1_ref, S_ref, T_ref):
    h = pl.program_id(0)
    xb = x_ref[...]
    f = jnp.maximum(jnp.dot(xb, W1_ref[0], precision=_HIGH) + b1_ref[h], 0.0)
    f = jnp.dot(f, W2_ref[0], precision=_HIGH) + b2_ref[h]
    S_ref[0] = jnp.dot(f, A1s_ref[0], precision=_HIGH) + a1_ref[h]
    pd = jnp.dot(f, A1d_ref[0], precision=_HIGH)
    T_ref[0] = jnp.concatenate([f, pd], axis=1)


def _prep(x, W1, b1, W2, b2, A1s, A1d, a1):
    grid = (H, N // BN)
    return pl.pallas_call(
        _prep_body,
        grid=grid,
        in_specs=[
            pl.BlockSpec((BN, D), lambda h, i: (i, 0)),
            pl.BlockSpec((1, D, D), lambda h, i: (h, 0, 0)),
            pl.BlockSpec((H, D), lambda h, i: (0, 0)),
            pl.BlockSpec((1, D, D), lambda h, i: (h, 0, 0)),
            pl.BlockSpec((H, D), lambda h, i: (0, 0)),
            pl.BlockSpec((1, D, AH), lambda h, i: (h, 0, 0)),
            pl.BlockSpec((1, D, AH), lambda h, i: (h, 0, 0)),
            pl.BlockSpec((H, AH), lambda h, i: (0, 0)),
        ],
        out_specs=[
            pl.BlockSpec((1, BN, AH), lambda h, i: (h, i, 0)),
            pl.BlockSpec((1, BN, R), lambda h, i: (h, i, 0)),
        ],
        out_shape=[
            jax.ShapeDtypeStruct((H, N, AH), jnp.float32),
            jax.ShapeDtypeStruct((H, N, R), jnp.float32),
        ],
    )(x, W1, b1, W2, b2, A1s, A1d, a1)


NISL = 4
NGSL = 3
NITER = (NCHUNK + 15) // 16
NUNROLL = 12
NOUTER = (NITER + NUNROLL) // NUNROLL


def _edge_body(S_hbm, T_hbm, srcr_hbm, dst_hbm, elem_hbm,
               consts_hbm, out_hbm, acc, cbuf, isrc, iga, igb, elv, g1v,
               g2v, scv, semA, semB, semS):
    h = lax.axis_index("c")
    tid = lax.axis_index("s")
    hN = h * N
    hE = h * E

    pltpu.sync_copy(consts_hbm.at[h], cbuf)
    c_vec = cbuf[0]
    a2v = cbuf[1]
    a2rep = cbuf[2]

    def _zrow(i, _):
        for k in range(R // 16):
            g2v[0][i, pl.ds(k * 16, 16)] = jnp.zeros((16,), jnp.float32)
        return 0

    lax.fori_loop(0, CH, _zrow, 0)

    def _zchunk(j, _):
        z = j * 16 + tid

        @pl.when(z < NZC)
        def _():
            pltpu.sync_copy(g2v[0], acc.at[pl.ds(z * CH, CH)])

        if ZTAIL:
            @pl.when(z == NZC)
            def _():
                pltpu.sync_copy(g2v[0].at[pl.ds(0, ZTAIL)],
                                acc.at[pl.ds(NZC * CH, ZTAIL)])

        return 0

    lax.fori_loop(0, (NZC + 16) // 16, _zchunk, 0)
    plsc.subcore_barrier()

    def _valid(j):
        return (j * 16 + tid) < NCHUNK

    def _base(j):
        return (j * 16 + tid) * CH

    def _issue_a(j, s):
        b = _base(j)
        pltpu.async_copy(srcr_hbm.at[pl.ds(b, CH)], isrc[s], semA[s])
        pltpu.async_copy(dst_hbm.at[pl.ds(b, CH)], igb[s], semA[s])
        pltpu.async_copy(elem_hbm.at[pl.ds(b, CH)],
                         elv[s].at[pl.ds(0, CH)], semA[s])

    def _wait_a(s):
        pltpu.make_async_copy(srcr_hbm.at[pl.ds(0, CH)], isrc[s], semA[s]).wait()
        pltpu.make_async_copy(dst_hbm.at[pl.ds(0, CH)], igb[s], semA[s]).wait()
        pltpu.make_async_copy(elem_hbm.at[pl.ds(0, CH)],
                              elv[s].at[pl.ds(0, CH)], semA[s]).wait()

    def _issue_b(si, sg):
        for k in range(CH // 16):
            sl = pl.ds(k * 16, 16)
            iga[si][sl] = isrc[si][sl] + hN
            igb[si][sl] = igb[si][sl] + hN
        pltpu.async_copy(S_hbm.at[iga[si]], g1v[sg], semB[sg])
        pltpu.async_copy(T_hbm.at[igb[si]], g2v[sg], semB[sg])

    def _wait_b(sg):
        pltpu.make_async_copy(S_hbm.at[pl.ds(0, CH)], g1v[sg], semB[sg]).wait()
        pltpu.make_async_copy(T_hbm.at[pl.ds(0, CH)], g2v[sg], semB[sg]).wait()

    def _issue_s(sg, si):
        pltpu.async_copy(g2v[sg], acc.at[isrc[si]], semS[sg], add=True)

    def _wait_s(sg):
        pltpu.make_async_copy(g2v[sg], acc.at[pl.ds(0, CH)], semS[sg]).wait()

    lane = lax.iota(jnp.int32, 16)

    def _compute(sg, si):
        def _group(g, _):
            g16 = g * 16
            el = elv[si][pl.ds(g16, 16)]
            row = lane + g16
            sc = a2rep
            for jj in range(AH):
                cj = jnp.full((16,), jj, jnp.int32)
                u = (plsc.load_gather(g1v[sg], [row, cj])
                     + plsc.load_gather(g2v[sg], [row, cj + D])
                     + el * c_vec[jj])
                sc = sc + jnp.maximum(u, 0.0) * a2v[jj]
            ev = jnp.exp(jnp.where(sc > 0, sc, 0.2 * sc))
            scv[pl.ds(g16, 16)] = ev
            return 0

        lax.fori_loop(0, CH // 16, _group, 0)

        def _scale(i, _):
            e = scv[pl.ds(i, 16)][0]
            for k in range(D // 16):
                sl = pl.ds(k * 16, 16)
                g2v[sg][i, sl] = g2v[sg][i, sl] * e
            g2v[sg][i, pl.ds(D, 16)] = jnp.where(lane == 0, e, 0.0)
            return 0

        lax.fori_loop(0, CH, _scale, 0)

    _issue_a(0, 0)
    _issue_a(1, 1)
    _issue_a(2, 2)
    _wait_a(0)
    _issue_b(0, 0)
    _wait_a(1)
    _issue_b(1, 1)

    def _iter(j, si, sg):
        @pl.when(_valid(j))
        def _():
            _wait_b(sg)
            _compute(sg, si)
            _issue_s(sg, si)

        @pl.when((j >= 1) & _valid(j - 1))
        def _():
            _wait_s((sg + 2) % NGSL)

        @pl.when(_valid(j + 3))
        def _():
            _issue_a(j + 3, (si + 3) % NISL)

        @pl.when(_valid(j + 2))
        def _():
            _wait_a((si + 2) % NISL)
            _issue_b((si + 2) % NISL, (sg + 2) % NGSL)

    def _outer(jj, _):
        j0 = jj * NUNROLL
        for b in range(NUNROLL):
            _iter(j0 + b, b % NISL, b % NGSL)
        return 0

    lax.fori_loop(0, NOUTER, _outer, 0)
    plsc.subcore_barrier()

    def _dump(j, _):
        z = j * 16 + tid

        @pl.when(z < NZC)
        def _():
            pltpu.sync_copy(acc.at[pl.ds(z * CH, CH)], g2v[0])
            pltpu.sync_copy(g2v[0], out_hbm.at[pl.ds(hN + z * CH, CH)])

        if ZTAIL:
            @pl.when(z == NZC)
            def _():
                pltpu.sync_copy(acc.at[pl.ds(NZC * CH, ZTAIL)],
                                g2v[0].at[pl.ds(0, ZTAIL)])
                pltpu.sync_copy(g2v[0].at[pl.ds(0, ZTAIL)],
                                out_hbm.at[pl.ds(hN + NZC * CH, ZTAIL)])

        return 0

    lax.fori_loop(0, (NZC + 16) // 16, _dump, 0)


def _edge_pass(S2, T2, srcr, dstr, elem, consts):
    mesh = plsc.VectorSubcoreMesh(core_axis_name="c", subcore_axis_name="s")
    k = pl.kernel(
        _edge_body,
        out_type=jax.ShapeDtypeStruct((H * N, R), jnp.float32),
        mesh=mesh,
        compiler_params=pltpu.CompilerParams(
            needs_layout_passes=False, use_tc_tiling_on_sc=False),
        scratch_types=[
            pltpu.VMEM_SHARED((N, R), jnp.float32),
            pltpu.VMEM((4, 16), jnp.float32),
            tuple(pltpu.VMEM((CH,), jnp.int32) for _ in range(NISL)),
            tuple(pltpu.VMEM((CH,), jnp.int32) for _ in range(NISL)),
            tuple(pltpu.VMEM((CH,), jnp.int32) for _ in range(NISL)),
            tuple(pltpu.VMEM((CH + 16,), jnp.float32) for _ in range(NISL)),
            tuple(pltpu.VMEM((CH, AH), jnp.float32) for _ in range(NGSL)),
            tuple(pltpu.VMEM((CH, R), jnp.float32) for _ in range(NGSL)),
            pltpu.VMEM((CH + 16,), jnp.float32),
            tuple(pltpu.SemaphoreType.DMA for _ in range(NISL)),
            tuple(pltpu.SemaphoreType.DMA for _ in range(NGSL)),
            tuple(pltpu.SemaphoreType.DMA for _ in range(NGSL)),
        ],
    )
    return k(S2, T2, srcr, dstr, elem, consts)


def _fin_body(P_ref, o_ref):
    pb = P_ref[0]
    o_ref[...] = pb[:, :D] / (pb[:, D:D + 1] + 1e-10)


def _finalize(P):
    grid = (H, N // BN)
    return pl.pallas_call(
        _fin_body,
        grid=grid,
        in_specs=[pl.BlockSpec((1, BN, R), lambda h, i: (h, i, 0))],
        out_specs=pl.BlockSpec((BN, D), lambda h, i: (i, h)),
        out_shape=jax.ShapeDtypeStruct((N, H * D), jnp.float32),
    )(P)


def kernel(x, idx, elem, W1, b1, W2, b2, A1, a1, A2, a2):
    A1s = A1[:, :D, :]
    A1d = A1[:, D:2 * D, :]
    c = A1[:, 2 * D, :]
    a2v = A2[:, :, 0]
    a2rep = jnp.broadcast_to(a2, (H, 16))
    consts = jnp.stack([c, a2v, a2rep, jnp.zeros_like(c)], axis=1)

    S, T = _prep(x, W1, b1, W2, b2, A1s, A1d, a1)
    S2 = S.reshape(H * N, AH)
    T2 = T.reshape(H * N, R)
    src = idx[0]
    dst = idx[1]
    P = _edge_pass(S2, T2, src, dst, elem, consts)
    return _finalize(P.reshape(H, N, R))


# speedup vs baseline: 1.8749x; 1.0462x over previous
import functools

import jax
import jax.numpy as jnp
from jax import lax
from jax.experimental import pallas as pl
from jax.experimental.pallas import tpu as pltpu
from jax.experimental.pallas import tpu_sc as plsc

N = 10000
D = 128
H = 2
AH = 16
E = 320000
R = 144
CH = 80
NCHUNK = E // CH
NZC = N // CH
ZTAIL = N - NZC * CH
BN = 1000

_HIGH = lax.Precision.HIGHEST


def _prep_body(x_ref, W1_ref, b1_ref, W2_ref, b2_ref, A1s_ref, A1d_ref,
               a1_ref, S_ref, T_ref):
    h = pl.program_id(0)
    xb = x_ref[...]
    f = jnp.maximum(jnp.dot(xb, W1_ref[0], precision=_HIGH) + b1_ref[h], 0.0)
    f = jnp.dot(f, W2_ref[0], precision=_HIGH) + b2_ref[h]
    S_ref[0] = jnp.dot(f, A1s_ref[0], precision=_HIGH) + a1_ref[h]
    pd = jnp.dot(f, A1d_ref[0], precision=_HIGH)
    T_ref[0] = jnp.concatenate([f, pd], axis=1)


def _prep(x, W1, b1, W2, b2, A1s, A1d, a1):
    grid = (H, N // BN)
    return pl.pallas_call(
        _prep_body,
        grid=grid,
        in_specs=[
            pl.BlockSpec((BN, D), lambda h, i: (i, 0)),
            pl.BlockSpec((1, D, D), lambda h, i: (h, 0, 0)),
            pl.BlockSpec((H, D), lambda h, i: (0, 0)),
            pl.BlockSpec((1, D, D), lambda h, i: (h, 0, 0)),
            pl.BlockSpec((H, D), lambda h, i: (0, 0)),
            pl.BlockSpec((1, D, AH), lambda h, i: (h, 0, 0)),
            pl.BlockSpec((1, D, AH), lambda h, i: (h, 0, 0)),
            pl.BlockSpec((H, AH), lambda h, i: (0, 0)),
        ],
        out_specs=[
            pl.BlockSpec((1, BN, AH), lambda h, i: (h, i, 0)),
            pl.BlockSpec((1, BN, R), lambda h, i: (h, i, 0)),
        ],
        out_shape=[
            jax.ShapeDtypeStruct((H, N, AH), jnp.float32),
            jax.ShapeDtypeStruct((H, N, R), jnp.float32),
        ],
    )(x, W1, b1, W2, b2, A1s, A1d, a1)


NISL = 4
NGSL = 3
NITER = (NCHUNK + 15) // 16
NUNROLL = 12
NOUTER = (NITER + NUNROLL) // NUNROLL


def _edge_body(S_hbm, T_hbm, srcr_hbm, dst_hbm, elem_hbm,
               consts_hbm, out_hbm, acc, cbuf, isrc, iga, igb, elv, g1v,
               g2v, scv, semA, semB, semS):
    h = lax.axis_index("c")
    tid = lax.axis_index("s")
    hN = h * N
    hE = h * E

    pltpu.sync_copy(consts_hbm.at[h], cbuf)
    c_vec = cbuf[0]
    a2v = cbuf[1]
    a2rep = cbuf[2]

    def _zrow(i, _):
        for k in range(R // 16):
            g2v[0][i, pl.ds(k * 16, 16)] = jnp.zeros((16,), jnp.float32)
        return 0

    lax.fori_loop(0, CH, _zrow, 0)

    def _zchunk(j, _):
        z = j * 16 + tid

        @pl.when(z < NZC)
        def _():
            pltpu.sync_copy(g2v[0], acc.at[pl.ds(z * CH, CH)])

        if ZTAIL:
            @pl.when(z == NZC)
            def _():
                pltpu.sync_copy(g2v[0].at[pl.ds(0, ZTAIL)],
                                acc.at[pl.ds(NZC * CH, ZTAIL)])

        return 0

    lax.fori_loop(0, (NZC + 16) // 16, _zchunk, 0)
    plsc.subcore_barrier()

    def _valid(j):
        return (j * 16 + tid) < NCHUNK

    def _base(j):
        return (j * 16 + tid) * CH

    def _issue_a(j, s):
        b = _base(j)
        pltpu.async_copy(srcr_hbm.at[pl.ds(b, CH)], isrc[s], semA[s])
        pltpu.async_copy(dst_hbm.at[pl.ds(b, CH)], igb[s], semA[s])
        pltpu.async_copy(elem_hbm.at[pl.ds(b, CH)],
                         elv[s].at[pl.ds(0, CH)], semA[s])

    def _wait_a(s):
        pltpu.make_async_copy(srcr_hbm.at[pl.ds(0, CH)], isrc[s], semA[s]).wait()
        pltpu.make_async_copy(dst_hbm.at[pl.ds(0, CH)], igb[s], semA[s]).wait()
        pltpu.make_async_copy(elem_hbm.at[pl.ds(0, CH)],
                              elv[s].at[pl.ds(0, CH)], semA[s]).wait()

    def _issue_b(si, sg):
        for k in range(CH // 16):
            sl = pl.ds(k * 16, 16)
            iga[si][sl] = isrc[si][sl] + hN
            igb[si][sl] = igb[si][sl] + hN
        pltpu.async_copy(S_hbm.at[iga[si]], g1v[sg], semB[sg])
        pltpu.async_copy(T_hbm.at[igb[si]], g2v[sg], semB[sg])

    def _wait_b(sg):
        pltpu.make_async_copy(S_hbm.at[pl.ds(0, CH)], g1v[sg], semB[sg]).wait()
        pltpu.make_async_copy(T_hbm.at[pl.ds(0, CH)], g2v[sg], semB[sg]).wait()

    def _issue_s(sg, si):
        pltpu.async_copy(g2v[sg], acc.at[isrc[si]], semS[sg], add=True)

    def _wait_s(sg):
        pltpu.make_async_copy(g2v[sg], acc.at[pl.ds(0, CH)], semS[sg]).wait()

    lane = lax.iota(jnp.int32, 16)

    def _compute(sg, si):
        def _group(g, _):
            g16 = g * 16
            el = elv[si][pl.ds(g16, 16)]
            row = lane + g16
            sc = a2rep
            for jj in range(AH):
                cj = jnp.full((16,), jj, jnp.int32)
                u = (plsc.load_gather(g1v[sg], [row, cj])
                     + plsc.load_gather(g2v[sg], [row, cj + D])
                     + el * c_vec[jj])
                sc = sc + jnp.maximum(u, 0.0) * a2v[jj]
            ev = jnp.exp(jnp.where(sc > 0, sc, 0.2 * sc))
            scv[pl.ds(g16, 16)] = ev
            return 0

        lax.fori_loop(0, CH // 16, _group, 0)

        def _scale(i, _):
            e = scv[pl.ds(i, 16)][0]
            for k in range(D // 16):
                sl = pl.ds(k * 16, 16)
                g2v[sg][i, sl] = g2v[sg][i, sl] * e
            g2v[sg][i, pl.ds(D, 16)] = jnp.where(lane == 0, e, 0.0)
            return 0

        lax.fori_loop(0, CH, _scale, 0)

    _issue_a(0, 0)
    _issue_a(1, 1)
    _issue_a(2, 2)
    _wait_a(0)
    _issue_b(0, 0)
    _wait_a(1)
    _issue_b(1, 1)

    def _iter(j, si, sg):
        @pl.when(_valid(j))
        def _():
            _wait_b(sg)
            _compute(sg, si)
            _issue_s(sg, si)

        @pl.when((j >= 1) & _valid(j - 1))
        def _():
            _wait_s((sg + 2) % NGSL)

        @pl.when(_valid(j + 3))
        def _():
            _issue_a(j + 3, (si + 3) % NISL)

        @pl.when(_valid(j + 2))
        def _():
            _wait_a((si + 2) % NISL)
            _issue_b((si + 2) % NISL, (sg + 2) % NGSL)

    def _outer(jj, _):
        j0 = jj * NUNROLL
        for b in range(NUNROLL):
            _iter(j0 + b, b % NISL, b % NGSL)
        return 0

    lax.fori_loop(0, NOUTER, _outer, 0)
    plsc.subcore_barrier()

    def _divrows(n):
        def _divrow(i, _):
            rsv = g2v[0][i, pl.ds(D, 16)]
            rinv = (1.0 / (rsv + 1e-10))[0]
            for k in range(D // 16):
                sl = pl.ds(k * 16, 16)
                g2v[0][i, sl] = g2v[0][i, sl] * rinv
            return 0

        lax.fori_loop(0, n, _divrow, 0)

    def _dump(j, _):
        z = j * 16 + tid

        @pl.when(z < NZC)
        def _():
            pltpu.sync_copy(acc.at[pl.ds(z * CH, CH)], g2v[0])
            _divrows(CH)
            pltpu.sync_copy(g2v[0].at[pl.ds(0, CH), pl.ds(0, D)],
                            out_hbm.at[pl.ds(z * CH, CH),
                                       pl.ds(h * D, D)])

        if ZTAIL:
            @pl.when(z == NZC)
            def _():
                pltpu.sync_copy(acc.at[pl.ds(NZC * CH, ZTAIL)],
                                g2v[0].at[pl.ds(0, ZTAIL)])
                _divrows(ZTAIL)
                pltpu.sync_copy(g2v[0].at[pl.ds(0, ZTAIL), pl.ds(0, D)],
                                out_hbm.at[pl.ds(NZC * CH, ZTAIL),
                                           pl.ds(h * D, D)])

        return 0

    lax.fori_loop(0, (NZC + 16) // 16, _dump, 0)


def _edge_pass(S2, T2, srcr, dstr, elem, consts):
    mesh = plsc.VectorSubcoreMesh(core_axis_name="c", subcore_axis_name="s")
    k = pl.kernel(
        _edge_body,
        out_type=jax.ShapeDtypeStruct((N, H * D), jnp.float32),
        mesh=mesh,
        compiler_params=pltpu.CompilerParams(
            needs_layout_passes=False, use_tc_tiling_on_sc=False),
        scratch_types=[
            pltpu.VMEM_SHARED((N, R), jnp.float32),
            pltpu.VMEM((4, 16), jnp.float32),
            tuple(pltpu.VMEM((CH,), jnp.int32) for _ in range(NISL)),
            tuple(pltpu.VMEM((CH,), jnp.int32) for _ in range(NISL)),
            tuple(pltpu.VMEM((CH,), jnp.int32) for _ in range(NISL)),
            tuple(pltpu.VMEM((CH + 16,), jnp.float32) for _ in range(NISL)),
            tuple(pltpu.VMEM((CH, AH), jnp.float32) for _ in range(NGSL)),
            tuple(pltpu.VMEM((CH, R), jnp.float32) for _ in range(NGSL)),
            pltpu.VMEM((CH + 16,), jnp.float32),
            tuple(pltpu.SemaphoreType.DMA for _ in range(NISL)),
            tuple(pltpu.SemaphoreType.DMA for _ in range(NGSL)),
            tuple(pltpu.SemaphoreType.DMA for _ in range(NGSL)),
        ],
    )
    return k(S2, T2, srcr, dstr, elem, consts)


def _fin_body(P_ref, o_ref):
    pb = P_ref[0]
    o_ref[...] = pb[:, :D] / (pb[:, D:D + 1] + 1e-10)


def _finalize(P):
    grid = (H, N // BN)
    return pl.pallas_call(
        _fin_body,
        grid=grid,
        in_specs=[pl.BlockSpec((1, BN, R), lambda h, i: (h, i, 0))],
        out_specs=pl.BlockSpec((BN, D), lambda h, i: (i, h)),
        out_shape=jax.ShapeDtypeStruct((N, H * D), jnp.float32),
    )(P)


def kernel(x, idx, elem, W1, b1, W2, b2, A1, a1, A2, a2):
    A1s = A1[:, :D, :]
    A1d = A1[:, D:2 * D, :]
    c = A1[:, 2 * D, :]
    a2v = A2[:, :, 0]
    a2rep = jnp.broadcast_to(a2, (H, 16))
    consts = jnp.stack([c, a2v, a2rep, jnp.zeros_like(c)], axis=1)

    S, T = _prep(x, W1, b1, W2, b2, A1s, A1d, a1)
    S2 = S.reshape(H * N, AH)
    T2 = T.reshape(H * N, R)
    src = idx[0]
    dst = idx[1]
    return _edge_pass(S2, T2, src, dst, elem, consts)


# speedup vs baseline: 2.1576x; 1.1508x over previous
import functools

import jax
import jax.numpy as jnp
from jax import lax
from jax.experimental import pallas as pl
from jax.experimental.pallas import tpu as pltpu
from jax.experimental.pallas import tpu_sc as plsc

N = 10000
D = 128
H = 2
AH = 16
E = 320000
R = 144
CH = 80
NCHUNK = E // CH
NZC = N // CH
ZTAIL = N - NZC * CH
BN = 1000

_HIGH = lax.Precision.DEFAULT


def _prep_body(x_ref, W1_ref, b1_ref, W2_ref, b2_ref, A1s_ref, A1d_ref,
               a1_ref, S_ref, T_ref):
    h = pl.program_id(0)
    xb = x_ref[...]
    f = jnp.maximum(jnp.dot(xb, W1_ref[0], precision=_HIGH) + b1_ref[h], 0.0)
    f = jnp.dot(f, W2_ref[0], precision=_HIGH) + b2_ref[h]
    S_ref[0] = jnp.dot(f, A1s_ref[0], precision=_HIGH) + a1_ref[h]
    pd = jnp.dot(f, A1d_ref[0], precision=_HIGH)
    T_ref[0] = jnp.concatenate([f, pd], axis=1)


def _prep(x, W1, b1, W2, b2, A1s, A1d, a1):
    grid = (H, N // BN)
    return pl.pallas_call(
        _prep_body,
        grid=grid,
        in_specs=[
            pl.BlockSpec((BN, D), lambda h, i: (i, 0)),
            pl.BlockSpec((1, D, D), lambda h, i: (h, 0, 0)),
            pl.BlockSpec((H, D), lambda h, i: (0, 0)),
            pl.BlockSpec((1, D, D), lambda h, i: (h, 0, 0)),
            pl.BlockSpec((H, D), lambda h, i: (0, 0)),
            pl.BlockSpec((1, D, AH), lambda h, i: (h, 0, 0)),
            pl.BlockSpec((1, D, AH), lambda h, i: (h, 0, 0)),
            pl.BlockSpec((H, AH), lambda h, i: (0, 0)),
        ],
        out_specs=[
            pl.BlockSpec((1, BN, AH), lambda h, i: (h, i, 0)),
            pl.BlockSpec((1, BN, R), lambda h, i: (h, i, 0)),
        ],
        out_shape=[
            jax.ShapeDtypeStruct((H, N, AH), jnp.float32),
            jax.ShapeDtypeStruct((H, N, R), jnp.float32),
        ],
    )(x, W1, b1, W2, b2, A1s, A1d, a1)


NISL = 4
NGSL = 3
NITER = (NCHUNK + 15) // 16
NUNROLL = 12
NOUTER = (NITER + NUNROLL) // NUNROLL


def _edge_body(S_hbm, T_hbm, srcr_hbm, dst_hbm, elem_hbm,
               consts_hbm, out_hbm, acc, cbuf, isrc, iga, igb, elv, g1v,
               g2v, scv, semA, semB, semS):
    h = lax.axis_index("c")
    tid = lax.axis_index("s")
    hN = h * N
    hE = h * E

    pltpu.sync_copy(consts_hbm.at[h], cbuf)
    c_vec = cbuf[0]
    a2v = cbuf[1]
    a2rep = cbuf[2]

    def _zrow(i, _):
        for k in range(R // 16):
            g2v[0][i, pl.ds(k * 16, 16)] = jnp.zeros((16,), jnp.float32)
        return 0

    lax.fori_loop(0, CH, _zrow, 0)

    def _zchunk(j, _):
        z = j * 16 + tid

        @pl.when(z < NZC)
        def _():
            pltpu.sync_copy(g2v[0], acc.at[pl.ds(z * CH, CH)])

        if ZTAIL:
            @pl.when(z == NZC)
            def _():
                pltpu.sync_copy(g2v[0].at[pl.ds(0, ZTAIL)],
                                acc.at[pl.ds(NZC * CH, ZTAIL)])

        return 0

    lax.fori_loop(0, (NZC + 16) // 16, _zchunk, 0)
    plsc.subcore_barrier()

    def _valid(j):
        return (j * 16 + tid) < NCHUNK

    def _base(j):
        return (j * 16 + tid) * CH

    def _issue_a(j, s):
        b = _base(j)
        pltpu.async_copy(srcr_hbm.at[pl.ds(b, CH)], isrc[s], semA[s])
        pltpu.async_copy(dst_hbm.at[pl.ds(b, CH)], igb[s], semA[s])
        pltpu.async_copy(elem_hbm.at[pl.ds(b, CH)],
                         elv[s].at[pl.ds(0, CH)], semA[s])

    def _wait_a(s):
        pltpu.make_async_copy(srcr_hbm.at[pl.ds(0, CH)], isrc[s], semA[s]).wait()
        pltpu.make_async_copy(dst_hbm.at[pl.ds(0, CH)], igb[s], semA[s]).wait()
        pltpu.make_async_copy(elem_hbm.at[pl.ds(0, CH)],
                              elv[s].at[pl.ds(0, CH)], semA[s]).wait()

    def _issue_b(si, sg):
        for k in range(CH // 16):
            sl = pl.ds(k * 16, 16)
            iga[si][sl] = isrc[si][sl] + hN
            igb[si][sl] = igb[si][sl] + hN
        pltpu.async_copy(S_hbm.at[iga[si]], g1v[sg], semB[sg])
        pltpu.async_copy(T_hbm.at[igb[si]], g2v[sg], semB[sg])

    def _wait_b(sg):
        pltpu.make_async_copy(S_hbm.at[pl.ds(0, CH)], g1v[sg], semB[sg]).wait()
        pltpu.make_async_copy(T_hbm.at[pl.ds(0, CH)], g2v[sg], semB[sg]).wait()

    def _issue_s(sg, si):
        pltpu.async_copy(g2v[sg], acc.at[isrc[si]], semS[sg], add=True)

    def _wait_s(sg):
        pltpu.make_async_copy(g2v[sg], acc.at[pl.ds(0, CH)], semS[sg]).wait()

    lane = lax.iota(jnp.int32, 16)

    def _compute(sg, si):
        def _group(g, _):
            g16 = g * 16
            el = elv[si][pl.ds(g16, 16)]
            row = lane + g16
            sc = a2rep
            for jj in range(AH):
                cj = jnp.full((16,), jj, jnp.int32)
                u = (plsc.load_gather(g1v[sg], [row, cj])
                     + plsc.load_gather(g2v[sg], [row, cj + D])
                     + el * c_vec[jj])
                sc = sc + jnp.maximum(u, 0.0) * a2v[jj]
            ev = jnp.exp(jnp.where(sc > 0, sc, 0.2 * sc))
            scv[pl.ds(g16, 16)] = ev
            return 0

        lax.fori_loop(0, CH // 16, _group, 0)

        def _scale(i, _):
            e = scv[pl.ds(i, 16)][0]
            for k in range(D // 16):
                sl = pl.ds(k * 16, 16)
                g2v[sg][i, sl] = g2v[sg][i, sl] * e
            g2v[sg][i, pl.ds(D, 16)] = jnp.where(lane == 0, e, 0.0)
            return 0

        lax.fori_loop(0, CH, _scale, 0)

    _issue_a(0, 0)
    _issue_a(1, 1)
    _issue_a(2, 2)
    _wait_a(0)
    _issue_b(0, 0)
    _wait_a(1)
    _issue_b(1, 1)

    def _iter(j, si, sg):
        @pl.when(_valid(j))
        def _():
            _wait_b(sg)
            _compute(sg, si)
            _issue_s(sg, si)

        @pl.when((j >= 1) & _valid(j - 1))
        def _():
            _wait_s((sg + 2) % NGSL)

        @pl.when(_valid(j + 3))
        def _():
            _issue_a(j + 3, (si + 3) % NISL)

        @pl.when(_valid(j + 2))
        def _():
            _wait_a((si + 2) % NISL)
            _issue_b((si + 2) % NISL, (sg + 2) % NGSL)

    def _outer(jj, _):
        j0 = jj * NUNROLL
        for b in range(NUNROLL):
            _iter(j0 + b, b % NISL, b % NGSL)
        return 0

    lax.fori_loop(0, NOUTER, _outer, 0)
    plsc.subcore_barrier()

    def _divrows(n):
        def _divrow(i, _):
            rsv = g2v[0][i, pl.ds(D, 16)]
            rinv = (1.0 / (rsv + 1e-10))[0]
            for k in range(D // 16):
                sl = pl.ds(k * 16, 16)
                g2v[0][i, sl] = g2v[0][i, sl] * rinv
            return 0

        lax.fori_loop(0, n, _divrow, 0)

    def _dump(j, _):
        z = j * 16 + tid

        @pl.when(z < NZC)
        def _():
            pltpu.sync_copy(acc.at[pl.ds(z * CH, CH)], g2v[0])
            _divrows(CH)
            pltpu.sync_copy(g2v[0].at[pl.ds(0, CH), pl.ds(0, D)],
                            out_hbm.at[pl.ds(z * CH, CH),
                                       pl.ds(h * D, D)])

        if ZTAIL:
            @pl.when(z == NZC)
            def _():
                pltpu.sync_copy(acc.at[pl.ds(NZC * CH, ZTAIL)],
                                g2v[0].at[pl.ds(0, ZTAIL)])
                _divrows(ZTAIL)
                pltpu.sync_copy(g2v[0].at[pl.ds(0, ZTAIL), pl.ds(0, D)],
                                out_hbm.at[pl.ds(NZC * CH, ZTAIL),
                                           pl.ds(h * D, D)])

        return 0

    lax.fori_loop(0, (NZC + 16) // 16, _dump, 0)


def _edge_pass(S2, T2, srcr, dstr, elem, consts):
    mesh = plsc.VectorSubcoreMesh(core_axis_name="c", subcore_axis_name="s")
    k = pl.kernel(
        _edge_body,
        out_type=jax.ShapeDtypeStruct((N, H * D), jnp.float32),
        mesh=mesh,
        compiler_params=pltpu.CompilerParams(
            needs_layout_passes=False, use_tc_tiling_on_sc=False),
        scratch_types=[
            pltpu.VMEM_SHARED((N, R), jnp.float32),
            pltpu.VMEM((4, 16), jnp.float32),
            tuple(pltpu.VMEM((CH,), jnp.int32) for _ in range(NISL)),
            tuple(pltpu.VMEM((CH,), jnp.int32) for _ in range(NISL)),
            tuple(pltpu.VMEM((CH,), jnp.int32) for _ in range(NISL)),
            tuple(pltpu.VMEM((CH + 16,), jnp.float32) for _ in range(NISL)),
            tuple(pltpu.VMEM((CH, AH), jnp.float32) for _ in range(NGSL)),
            tuple(pltpu.VMEM((CH, R), jnp.float32) for _ in range(NGSL)),
            pltpu.VMEM((CH + 16,), jnp.float32),
            tuple(pltpu.SemaphoreType.DMA for _ in range(NISL)),
            tuple(pltpu.SemaphoreType.DMA for _ in range(NGSL)),
            tuple(pltpu.SemaphoreType.DMA for _ in range(NGSL)),
        ],
    )
    return k(S2, T2, srcr, dstr, elem, consts)


def _fin_body(P_ref, o_ref):
    pb = P_ref[0]
    o_ref[...] = pb[:, :D] / (pb[:, D:D + 1] + 1e-10)


def _finalize(P):
    grid = (H, N // BN)
    return pl.pallas_call(
        _fin_body,
        grid=grid,
        in_specs=[pl.BlockSpec((1, BN, R), lambda h, i: (h, i, 0))],
        out_specs=pl.BlockSpec((BN, D), lambda h, i: (i, h)),
        out_shape=jax.ShapeDtypeStruct((N, H * D), jnp.float32),
    )(P)


def kernel(x, idx, elem, W1, b1, W2, b2, A1, a1, A2, a2):
    A1s = A1[:, :D, :]
    A1d = A1[:, D:2 * D, :]
    c = A1[:, 2 * D, :]
    a2v = A2[:, :, 0]
    a2rep = jnp.broadcast_to(a2, (H, 16))
    consts = jnp.stack([c, a2v, a2rep, jnp.zeros_like(c)], axis=1)

    S, T = _prep(x, W1, b1, W2, b2, A1s, A1d, a1)
    S2 = S.reshape(H * N, AH)
    T2 = T.reshape(H * N, R)
    src = idx[0]
    dst = idx[1]
    return _edge_pass(S2, T2, src, dst, elem, consts)


# speedup vs baseline: 2.1578x; 1.0001x over previous
import jax
import jax.numpy as jnp
from jax import lax
from jax.experimental import pallas as pl
from jax.experimental.pallas import tpu as pltpu
from jax.experimental.pallas import tpu_sc as plsc

N = 10000
D = 128
H = 2
AH = 16
E = 320000
R = 144
CH = 80
NCHUNK = E // CH
NZC = N // CH
ZTAIL = N - NZC * CH
BN = 1000

_HIGH = lax.Precision.DEFAULT


def _prep_body(x_ref, W1_ref, b1_ref, W2_ref, b2_ref, A1s_ref, A1d_ref,
               a1_ref, S_ref, T_ref):
    h = pl.program_id(0)
    xb = x_ref[...]
    f = jnp.maximum(jnp.dot(xb, W1_ref[0], precision=_HIGH) + b1_ref[h], 0.0)
    f = jnp.dot(f, W2_ref[0], precision=_HIGH) + b2_ref[h]
    S_ref[0] = jnp.dot(f, A1s_ref[0], precision=_HIGH) + a1_ref[h]
    pd = jnp.dot(f, A1d_ref[0], precision=_HIGH)
    T_ref[0] = jnp.concatenate([f, pd], axis=1)


def _prep(x, W1, b1, W2, b2, A1s, A1d, a1):
    grid = (H, N // BN)
    return pl.pallas_call(
        _prep_body,
        grid=grid,
        in_specs=[
            pl.BlockSpec((BN, D), lambda h, i: (i, 0)),
            pl.BlockSpec((1, D, D), lambda h, i: (h, 0, 0)),
            pl.BlockSpec((H, D), lambda h, i: (0, 0)),
            pl.BlockSpec((1, D, D), lambda h, i: (h, 0, 0)),
            pl.BlockSpec((H, D), lambda h, i: (0, 0)),
            pl.BlockSpec((1, D, AH), lambda h, i: (h, 0, 0)),
            pl.BlockSpec((1, D, AH), lambda h, i: (h, 0, 0)),
            pl.BlockSpec((H, AH), lambda h, i: (0, 0)),
        ],
        out_specs=[
            pl.BlockSpec((1, BN, AH), lambda h, i: (h, i, 0)),
            pl.BlockSpec((1, BN, R), lambda h, i: (h, i, 0)),
        ],
        out_shape=[
            jax.ShapeDtypeStruct((H, N, AH), jnp.float32),
            jax.ShapeDtypeStruct((H, N, R), jnp.float32),
        ],
    )(x, W1, b1, W2, b2, A1s, A1d, a1)


NISL = 4
NGSL = 3
NITER = (NCHUNK + 15) // 16
NUNROLL = 12
NOUTER = (NITER + NUNROLL) // NUNROLL


def _edge_body(S_hbm, T_hbm, srcr_hbm, dst_hbm, elem_hbm,
               consts_hbm, out_hbm, acc, cbuf, isrc, iga, igb, elv, g1v,
               g2v, scv, semA, semB, semS):
    h = lax.axis_index("c")
    tid = lax.axis_index("s")
    hN = h * N
    hE = h * E

    pltpu.sync_copy(consts_hbm.at[h], cbuf)
    c_vec = cbuf[0]
    a2v = cbuf[1]
    a2rep = cbuf[2]

    def _zrow(i, _):
        for k in range(R // 16):
            g2v[0][i, pl.ds(k * 16, 16)] = jnp.zeros((16,), jnp.float32)
        return 0

    lax.fori_loop(0, CH, _zrow, 0)

    def _zchunk(j, _):
        z = j * 16 + tid

        @pl.when(z < NZC)
        def _():
            pltpu.sync_copy(g2v[0], acc.at[pl.ds(z * CH, CH)])

        if ZTAIL:
            @pl.when(z == NZC)
            def _():
                pltpu.sync_copy(g2v[0].at[pl.ds(0, ZTAIL)],
                                acc.at[pl.ds(NZC * CH, ZTAIL)])

        return 0

    lax.fori_loop(0, (NZC + 16) // 16, _zchunk, 0)
    plsc.subcore_barrier()

    def _valid(j):
        return (j * 16 + tid) < NCHUNK

    def _base(j):
        return (j * 16 + tid) * CH

    def _issue_a(j, s):
        b = _base(j)
        pltpu.async_copy(srcr_hbm.at[pl.ds(b, CH)], isrc[s], semA[s])
        pltpu.async_copy(dst_hbm.at[pl.ds(b, CH)], igb[s], semA[s])
        pltpu.async_copy(elem_hbm.at[pl.ds(b, CH)],
                         elv[s].at[pl.ds(0, CH)], semA[s])

    def _wait_a(s):
        pltpu.make_async_copy(srcr_hbm.at[pl.ds(0, CH)], isrc[s], semA[s]).wait()
        pltpu.make_async_copy(dst_hbm.at[pl.ds(0, CH)], igb[s], semA[s]).wait()
        pltpu.make_async_copy(elem_hbm.at[pl.ds(0, CH)],
                              elv[s].at[pl.ds(0, CH)], semA[s]).wait()

    def _issue_b(si, sg):
        for k in range(CH // 16):
            sl = pl.ds(k * 16, 16)
            iga[si][sl] = isrc[si][sl] + hN
            igb[si][sl] = igb[si][sl] + hN
        pltpu.async_copy(S_hbm.at[iga[si]], g1v[sg], semB[sg])
        pltpu.async_copy(T_hbm.at[igb[si]], g2v[sg], semB[sg])

    def _wait_b(sg):
        pltpu.make_async_copy(S_hbm.at[pl.ds(0, CH)], g1v[sg], semB[sg]).wait()
        pltpu.make_async_copy(T_hbm.at[pl.ds(0, CH)], g2v[sg], semB[sg]).wait()

    def _issue_s(sg, si):
        pltpu.async_copy(g2v[sg], acc.at[isrc[si]], semS[sg], add=True)

    def _wait_s(sg):
        pltpu.make_async_copy(g2v[sg], acc.at[pl.ds(0, CH)], semS[sg]).wait()

    lane = lax.iota(jnp.int32, 16)

    def _compute(sg, si):
        def _group(g, _):
            g16 = g * 16
            el = elv[si][pl.ds(g16, 16)]
            row = lane + g16
            sc = a2rep
            for jj in range(AH):
                cj = jnp.full((16,), jj, jnp.int32)
                u = (plsc.load_gather(g1v[sg], [row, cj])
                     + plsc.load_gather(g2v[sg], [row, cj + D])
                     + el * c_vec[jj])
                sc = sc + jnp.maximum(u, 0.0) * a2v[jj]
            ev = jnp.exp(jnp.where(sc > 0, sc, 0.2 * sc))
            scv[pl.ds(g16, 16)] = ev
            return 0

        lax.fori_loop(0, CH // 16, _group, 0)

        def _scale(i, _):
            e = scv[pl.ds(i, 16)][0]
            for k in range(D // 16):
                sl = pl.ds(k * 16, 16)
                g2v[sg][i, sl] = g2v[sg][i, sl] * e
            g2v[sg][i, pl.ds(D, 16)] = jnp.where(lane == 0, e, 0.0)
            return 0

        lax.fori_loop(0, CH, _scale, 0)

    _issue_a(0, 0)
    _issue_a(1, 1)
    _issue_a(2, 2)
    _wait_a(0)
    _issue_b(0, 0)
    _wait_a(1)
    _issue_b(1, 1)

    def _iter(j, si, sg):
        @pl.when(_valid(j))
        def _():
            _wait_b(sg)
            _compute(sg, si)
            _issue_s(sg, si)

        @pl.when((j >= 1) & _valid(j - 1))
        def _():
            _wait_s((sg + 2) % NGSL)

        @pl.when(_valid(j + 3))
        def _():
            _issue_a(j + 3, (si + 3) % NISL)

        @pl.when(_valid(j + 2))
        def _():
            _wait_a((si + 2) % NISL)
            _issue_b((si + 2) % NISL, (sg + 2) % NGSL)

    def _outer(jj, _):
        j0 = jj * NUNROLL
        for b in range(NUNROLL):
            _iter(j0 + b, b % NISL, b % NGSL)
        return 0

    lax.fori_loop(0, NOUTER, _outer, 0)
    plsc.subcore_barrier()

    def _divrows(n):
        def _divrow(i, _):
            rsv = g2v[0][i, pl.ds(D, 16)]
            rinv = (1.0 / (rsv + 1e-10))[0]
            for k in range(D // 16):
                sl = pl.ds(k * 16, 16)
                g2v[0][i, sl] = g2v[0][i, sl] * rinv
            return 0

        lax.fori_loop(0, n, _divrow, 0)

    def _dump(j, _):
        z = j * 16 + tid

        @pl.when(z < NZC)
        def _():
            pltpu.sync_copy(acc.at[pl.ds(z * CH, CH)], g2v[0])
            _divrows(CH)
            pltpu.sync_copy(g2v[0].at[pl.ds(0, CH), pl.ds(0, D)],
                            out_hbm.at[pl.ds(z * CH, CH),
                                       pl.ds(h * D, D)])

        if ZTAIL:
            @pl.when(z == NZC)
            def _():
                pltpu.sync_copy(acc.at[pl.ds(NZC * CH, ZTAIL)],
                                g2v[0].at[pl.ds(0, ZTAIL)])
                _divrows(ZTAIL)
                pltpu.sync_copy(g2v[0].at[pl.ds(0, ZTAIL), pl.ds(0, D)],
                                out_hbm.at[pl.ds(NZC * CH, ZTAIL),
                                           pl.ds(h * D, D)])

        return 0

    lax.fori_loop(0, (NZC + 16) // 16, _dump, 0)


def _edge_pass(S2, T2, srcr, dstr, elem, consts):
    mesh = plsc.VectorSubcoreMesh(core_axis_name="c", subcore_axis_name="s")
    k = pl.kernel(
        _edge_body,
        out_type=jax.ShapeDtypeStruct((N, H * D), jnp.float32),
        mesh=mesh,
        compiler_params=pltpu.CompilerParams(
            needs_layout_passes=False, use_tc_tiling_on_sc=False),
        scratch_types=[
            pltpu.VMEM_SHARED((N, R), jnp.float32),
            pltpu.VMEM((4, 16), jnp.float32),
            tuple(pltpu.VMEM((CH,), jnp.int32) for _ in range(NISL)),
            tuple(pltpu.VMEM((CH,), jnp.int32) for _ in range(NISL)),
            tuple(pltpu.VMEM((CH,), jnp.int32) for _ in range(NISL)),
            tuple(pltpu.VMEM((CH + 16,), jnp.float32) for _ in range(NISL)),
            tuple(pltpu.VMEM((CH, AH), jnp.float32) for _ in range(NGSL)),
            tuple(pltpu.VMEM((CH, R), jnp.float32) for _ in range(NGSL)),
            pltpu.VMEM((CH + 16,), jnp.float32),
            tuple(pltpu.SemaphoreType.DMA for _ in range(NISL)),
            tuple(pltpu.SemaphoreType.DMA for _ in range(NGSL)),
            tuple(pltpu.SemaphoreType.DMA for _ in range(NGSL)),
        ],
    )
    return k(S2, T2, srcr, dstr, elem, consts)


def kernel(x, idx, elem, W1, b1, W2, b2, A1, a1, A2, a2):
    A1s = A1[:, :D, :]
    A1d = A1[:, D:2 * D, :]
    c = A1[:, 2 * D, :]
    a2v = A2[:, :, 0]
    a2rep = jnp.broadcast_to(a2, (H, 16))
    consts = jnp.stack([c, a2v, a2rep, jnp.zeros_like(c)], axis=1)

    S, T = _prep(x, W1, b1, W2, b2, A1s, A1d, a1)
    S2 = S.reshape(H * N, AH)
    T2 = T.reshape(H * N, R)
    src = idx[0]
    dst = idx[1]
    return _edge_pass(S2, T2, src, dst, elem, consts)
